# baseline (speedup 1.0000x reference)
import functools

import jax
import jax.numpy as jnp
from jax import lax
from jax.experimental import pallas as pl
from jax.experimental.pallas import tpu as pltpu
from jax.experimental.pallas import tpu_sc as plsc

NUM_ENTITIES = 1000000
EMBEDDING_DIM = 32
BATCH = 16384

_LANE = 128

_info = plsc.get_sparse_core_info()
_NC, _NS = _info.num_cores, _info.num_subcores
_NW = _NC * _NS
_B_PER_W = BATCH // _NW
_NBUF = 24


def _scalar_at(idx_v, lane_i, k):
    vec = idx_v[pl.ds(pl.multiple_of((k >> 4) * 16, 16), 16)]
    return lax.reduce_max(jnp.where(lane_i == (k & 15), vec, 0), (0,))


def _fetch(tab_hbm, idx_v, lane_i, buf, sem, k):
    tc = _scalar_at(idx_v, lane_i, k) >> 7
    return pltpu.make_async_copy(
        tab_hbm.at[:, pl.ds(pl.multiple_of(tc * _LANE, _LANE), _LANE)],
        buf,
        sem,
    )


def _gather_body(idx_hbm, tab_hbm, out_t_hbm, idx_v, bufs, block_v, sems):
    wid = lax.axis_index("s") * _NC + lax.axis_index("c")
    base = wid * _B_PER_W
    pltpu.sync_copy(idx_hbm.at[pl.ds(base, _B_PER_W)], idx_v)
    lane_i = jax.lax.broadcasted_iota(jnp.int32, (16,), 0)

    for b in range(_NBUF):
        _fetch(tab_hbm, idx_v, lane_i, bufs[b], sems[b % 4], b).start()

    def step(g, carry):
        del carry
        for b in range(_NBUF):
            k = g * _NBUF + b

            @pl.when(k < _B_PER_W)
            def _():
                _fetch(tab_hbm, idx_v, lane_i, bufs[b], sems[b % 4], k).wait()
                c = _scalar_at(idx_v, lane_i, k) & (_LANE - 1)
                c_vec = jnp.full((16,), c, jnp.int32)
                k_vec = jnp.full((16,), k, jnp.int32)
                for h in range(2):
                    vals = plsc.load_gather(bufs[b], [16 * h + lane_i, c_vec])
                    plsc.store_scatter(block_v, [16 * h + lane_i, k_vec], vals)

                @pl.when(k + _NBUF < _B_PER_W)
                def _():
                    _fetch(
                        tab_hbm, idx_v, lane_i, bufs[b], sems[b % 4], k + _NBUF
                    ).start()

        return 0

    lax.fori_loop(0, -(-_B_PER_W // _NBUF), step, 0)
    pltpu.sync_copy(block_v, out_t_hbm.at[:, pl.ds(base, _B_PER_W)])


@jax.jit
def kernel(elements, entity_embeddings):
    mesh = plsc.VectorSubcoreMesh(core_axis_name="c", subcore_axis_name="s")
    gather = pl.kernel(
        _gather_body,
        mesh=mesh,
        out_type=jax.ShapeDtypeStruct((EMBEDDING_DIM, BATCH), jnp.float32),
        scratch_types=[
            pltpu.VMEM((_B_PER_W,), jnp.int32),
            [pltpu.VMEM((EMBEDDING_DIM, _LANE), jnp.float32) for _ in range(_NBUF)],
            pltpu.VMEM((EMBEDDING_DIM, _B_PER_W), jnp.float32),
            [pltpu.SemaphoreType.DMA for _ in range(4)],
        ],
        compiler_params=pltpu.CompilerParams(needs_layout_passes=False),
    )
    out_t = gather(elements.astype(jnp.int32), entity_embeddings.T)
    return out_t.T


# speedup vs baseline: 1.0433x; 1.0433x over previous
import functools

import jax
import jax.numpy as jnp
from jax import lax
from jax.experimental import pallas as pl
from jax.experimental.pallas import tpu as pltpu
from jax.experimental.pallas import tpu_sc as plsc

NUM_ENTITIES = 1000000
EMBEDDING_DIM = 32
BATCH = 16384

_LANE = 128

_info = plsc.get_sparse_core_info()
_NC, _NS = _info.num_cores, _info.num_subcores
_NW = _NC * _NS
_B_PER_W = BATCH // _NW
_NBUF = 16


def _scalar_at(idx_v, lane_i, k):
    vec = idx_v[pl.ds(pl.multiple_of((k >> 4) * 16, 16), 16)]
    return lax.reduce_max(jnp.where(lane_i == (k & 15), vec, 0), (0,))


def _fetch(tab_hbm, idx_v, lane_i, buf, sem, k):
    tc = _scalar_at(idx_v, lane_i, k) >> 7
    return pltpu.make_async_copy(
        tab_hbm.at[:, pl.ds(pl.multiple_of(tc * _LANE, _LANE), _LANE)],
        buf,
        sem,
    )


def _gather_body(idx_hbm, tab_hbm, out_t_hbm, idx_v, bufs, block_v, sems):
    wid = lax.axis_index("s") * _NC + lax.axis_index("c")
    base = wid * _B_PER_W
    pltpu.sync_copy(idx_hbm.at[pl.ds(base, _B_PER_W)], idx_v)
    lane_i = jax.lax.broadcasted_iota(jnp.int32, (16,), 0)

    for b in range(_NBUF):
        _fetch(tab_hbm, idx_v, lane_i, bufs[b], sems[b % 4], b).start()

    def step(g, carry):
        del carry
        for b in range(_NBUF):
            k = g * _NBUF + b
            _fetch(tab_hbm, idx_v, lane_i, bufs[b], sems[b % 4], k).wait()
            c = _scalar_at(idx_v, lane_i, k) & (_LANE - 1)
            c_vec = jnp.full((16,), c, jnp.int32)
            k_vec = jnp.full((16,), k, jnp.int32)
            for h in range(2):
                vals = plsc.load_gather(bufs[b], [16 * h + lane_i, c_vec])
                plsc.store_scatter(block_v, [16 * h + lane_i, k_vec], vals)

            @pl.when(k + _NBUF < _B_PER_W)
            def _():
                _fetch(
                    tab_hbm, idx_v, lane_i, bufs[b], sems[b % 4], k + _NBUF
                ).start()

        return 0

    lax.fori_loop(0, _B_PER_W // _NBUF, step, 0)
    pltpu.sync_copy(block_v, out_t_hbm.at[:, pl.ds(base, _B_PER_W)])


@jax.jit
def kernel(elements, entity_embeddings):
    mesh = plsc.VectorSubcoreMesh(core_axis_name="c", subcore_axis_name="s")
    gather = pl.kernel(
        _gather_body,
        mesh=mesh,
        out_type=jax.ShapeDtypeStruct((EMBEDDING_DIM, BATCH), jnp.float32),
        scratch_types=[
            pltpu.VMEM((_B_PER_W,), jnp.int32),
            [pltpu.VMEM((EMBEDDING_DIM, _LANE), jnp.float32) for _ in range(_NBUF)],
            pltpu.VMEM((EMBEDDING_DIM, _B_PER_W), jnp.float32),
            [pltpu.SemaphoreType.DMA for _ in range(4)],
        ],
        compiler_params=pltpu.CompilerParams(needs_layout_passes=False),
    )
    out_t = gather(elements.astype(jnp.int32), entity_embeddings.T)
    return out_t.T


# speedup vs baseline: 1.0504x; 1.0068x over previous
import jax
import jax.numpy as jnp
from jax import lax
from jax.experimental import pallas as pl
from jax.experimental.pallas import tpu as pltpu
from jax.experimental.pallas import tpu_sc as plsc

NUM_ENTITIES = 1000000
EMBEDDING_DIM = 32
BATCH = 16384

_LANE = 128

_info = plsc.get_sparse_core_info()
_NC, _NS = _info.num_cores, _info.num_subcores
_NW = _NC * _NS
_B_PER_W = BATCH // _NW
_NBUF = 16


def _scalar_at(idx_v, lane_i, k):
    vec = idx_v[pl.ds(pl.multiple_of((k >> 4) * 16, 16), 16)]
    return lax.reduce_max(jnp.where(lane_i == (k & 15), vec, 0), (0,))


def _fetch(tab_hbm, idx_v, lane_i, buf, sem, k):
    tc = _scalar_at(idx_v, lane_i, k) >> 7
    return pltpu.make_async_copy(
        tab_hbm.at[:, pl.ds(pl.multiple_of(tc * _LANE, _LANE), _LANE)],
        buf,
        sem,
    )


def _gather_body(idx_hbm, tab_hbm, out_t_hbm, idx_v, bufs, block_v, sems):
    wid = lax.axis_index("s") * _NC + lax.axis_index("c")
    base = wid * _B_PER_W
    pltpu.sync_copy(idx_hbm.at[pl.ds(base, _B_PER_W)], idx_v)
    lane_i = jax.lax.broadcasted_iota(jnp.int32, (16,), 0)

    for b in range(_NBUF):
        _fetch(tab_hbm, idx_v, lane_i, bufs[b], sems[b % 4], b).start()

    def step(g, carry):
        del carry
        for b in range(_NBUF):
            k = g * _NBUF + b
            _fetch(tab_hbm, idx_v, lane_i, bufs[b], sems[b % 4], k).wait()
            c = _scalar_at(idx_v, lane_i, k) & (_LANE - 1)
            c_vec = jnp.full((16,), c, jnp.int32)
            k_vec = jnp.full((16,), k, jnp.int32)
            for h in range(2):
                vals = plsc.load_gather(bufs[b], [16 * h + lane_i, c_vec])
                plsc.store_scatter(block_v, [16 * h + lane_i, k_vec], vals)

            @pl.when(k + _NBUF < _B_PER_W)
            def _():
                _fetch(
                    tab_hbm, idx_v, lane_i, bufs[b], sems[b % 4], k + _NBUF
                ).start()

        return 0

    lax.fori_loop(0, _B_PER_W // _NBUF, step, 0)
    pltpu.sync_copy(block_v, out_t_hbm.at[:, pl.ds(base, _B_PER_W)])


@jax.jit
def kernel(elements, entity_embeddings):
    mesh = plsc.VectorSubcoreMesh(core_axis_name="c", subcore_axis_name="s")
    gather = pl.kernel(
        _gather_body,
        mesh=mesh,
        out_type=jax.ShapeDtypeStruct((EMBEDDING_DIM, BATCH), jnp.float32),
        scratch_types=[
            pltpu.VMEM((_B_PER_W,), jnp.int32),
            [pltpu.VMEM((EMBEDDING_DIM, _LANE), jnp.float32) for _ in range(_NBUF)],
            pltpu.VMEM((EMBEDDING_DIM, _B_PER_W), jnp.float32),
            [pltpu.SemaphoreType.DMA for _ in range(4)],
        ],
        compiler_params=pltpu.CompilerParams(needs_layout_passes=False),
    )
    out_t = gather(elements.astype(jnp.int32), entity_embeddings.T)
    return out_t.T
